# baseline (device time: 38056 ns/iter reference)
import jax
import jax.numpy as jnp
from jax import lax
from jax.experimental import pallas as pl
from jax.experimental.pallas import tpu as pltpu

N_DEV = 8
E_LOC = 8
T = 2048
D = 512
H = 1024
TC = T // N_DEV
CAP = 128
ABLATE_NO_COMM = False


def kernel(x, router_W, route_idx, expert_W):
    def body(x_ref, rw_ref, idx_ref, ew_ref, out_ref,
             wbf_ref, xsc_ref, coef_ref, send_buf, recv_buf, ew_vmem,
             send_sems, recv_sems, load_sems):
        d = lax.axis_index("i")

        ldmas = []
        for j in range(E_LOC):
            ldma = pltpu.make_async_copy(
                ew_ref.at[j], ew_vmem.at[j], load_sems.at[j])
            ldma.start()
            ldmas.append(ldma)

        with jax.named_scope("entrybarrier"):
            if not ABLATE_NO_COMM:
                barrier_sem = pltpu.get_barrier_semaphore()
                for k in range(1, N_DEV):
                    pl.semaphore_signal(
                        barrier_sem, inc=1,
                        device_id=(jnp.mod(d + k, N_DEV),),
                        device_id_type=pl.DeviceIdType.MESH,
                    )
                pl.semaphore_wait(barrier_sem, N_DEV - 1)

        with jax.named_scope("gating"):
            xb = x_ref[...].astype(jnp.bfloat16)
            scores = jnp.dot(xb, rw_ref[...].astype(jnp.bfloat16),
                             preferred_element_type=jnp.float32)
            idx0 = idx_ref[:, 0:1]
            idx1 = idx_ref[:, 1:2]
            e_iota = lax.broadcasted_iota(jnp.int32, scores.shape, 1)
            s0 = jnp.sum(jnp.where(e_iota == idx0, scores, 0.0), axis=1,
                         keepdims=True)
            s1 = jnp.sum(jnp.where(e_iota == idx1, scores, 0.0), axis=1,
                         keepdims=True)
            g0 = jax.nn.sigmoid(s0 - s1)
            g1 = 1.0 - g0
            gids = d * E_LOC + lax.broadcasted_iota(jnp.int32, (T, E_LOC), 1)
            coef_ref[...] = (jnp.where(idx0 == gids, g0, 0.0)
                             + jnp.where(idx1 == gids, g1, 0.0))

        slot_iota = lax.broadcasted_iota(jnp.int32, (TC, CAP), 1)
        row_col_bf = lax.broadcasted_iota(
            jnp.int32, (TC, 1), 0).astype(jnp.bfloat16)
        ia = lax.broadcasted_iota(jnp.int32, (TC, TC), 0)
        ib = lax.broadcasted_iota(jnp.int32, (TC, TC), 1)
        tril = (ib < ia).astype(jnp.bfloat16)

        def build_pt(c):
            i0 = idx_ref[pl.ds(c * TC, TC), 0:1]
            i1 = idx_ref[pl.ds(c * TC, TC), 1:2]
            act = jnp.logical_or(i0 // E_LOC == d, i1 // E_LOC == d)
            rank = jnp.dot(tril, act.astype(jnp.bfloat16),
                           preferred_element_type=jnp.float32)
            return jnp.logical_and(
                rank.astype(jnp.int32) == slot_iota, act
            ).astype(jnp.bfloat16)

        def gather_into(c, pt, half):
            cc = coef_ref[pl.ds(c * TC, TC), :]
            xc = x_ref[pl.ds(c * TC, TC), :].astype(jnp.bfloat16)
            xg = lax.dot_general(
                pt, xc, (((0,), (0,)), ((), ())),
                preferred_element_type=jnp.float32)
            ccomp = lax.dot_general(
                pt, cc.astype(jnp.bfloat16), (((0,), (0,)), ((), ())),
                preferred_element_type=jnp.float32)
            for j in range(E_LOC):
                xsc_ref[CAP * half:CAP * (half + 1), D * j:D * (j + 1)] = (
                    ccomp[:, j:j + 1] * xg).astype(jnp.bfloat16)

        def make_msg(pt, payload):
            ids = lax.dot_general(
                row_col_bf, pt,
                (((0,), (0,)), ((), ())),
                preferred_element_type=jnp.float32)
            ids_row = jnp.concatenate(
                [ids, jnp.zeros((1, H - CAP), jnp.float32)], axis=1)
            return jnp.concatenate(
                [ids_row, payload], axis=0).astype(jnp.bfloat16)

        recv_row_iota = lax.broadcasted_iota(jnp.int32, (TC, CAP), 0)

        def scatter_recv(k, racc):
            rdmas[k - 1].wait_recv()
            ids = recv_buf[k - 1, 0:1, 0:CAP].astype(jnp.int32)
            scatter = (recv_row_iota == ids).astype(jnp.bfloat16)
            contrib = jnp.dot(scatter, recv_buf[k - 1, 1:1 + CAP, :],
                              preferred_element_type=jnp.float32)
            return contrib if racc is None else racc + contrib

        rdmas = [None] * (N_DEV - 1)
        acc = None
        racc = None
        ks = [1, 2, 3, 4, 5, 6, 7, 0]
        for p in range(N_DEV // 2):
            with jax.named_scope(f"pair{p}"):
                pair = ks[2 * p:2 * p + 2]
                pts = []
                for half, k in enumerate(pair):
                    c = jnp.mod(d + k, N_DEV)
                    pt = build_pt(c)
                    gather_into(c, pt, half)
                    pts.append(pt)
                if p == 0:
                    with jax.named_scope("precast"):
                        for j in range(E_LOC):
                            ldmas[j].wait()
                            wbf_ref[pl.ds(D * j, D), :] = (
                                ew_vmem[j].astype(jnp.bfloat16))
                y = jnp.dot(xsc_ref[...], wbf_ref[...],
                            preferred_element_type=jnp.float32)
                for half, k in enumerate(pair):
                    payload = y[CAP * half:CAP * (half + 1), :]
                    if k == 0:
                        acc = jnp.dot(pts[half], payload.astype(jnp.bfloat16),
                                      preferred_element_type=jnp.float32)
                        continue
                    send_buf[k - 1] = make_msg(pts[half], payload)
                    if not ABLATE_NO_COMM:
                        rdma = pltpu.make_async_remote_copy(
                            src_ref=send_buf.at[k - 1],
                            dst_ref=recv_buf.at[k - 1],
                            send_sem=send_sems.at[k - 1],
                            recv_sem=recv_sems.at[k - 1],
                            device_id=(jnp.mod(d + k, N_DEV),),
                            device_id_type=pl.DeviceIdType.MESH,
                        )
                        rdma.start()
                        rdmas[k - 1] = rdma

        with jax.named_scope("waitadd"):
            if not ABLATE_NO_COMM:
                for k in range(1, N_DEV):
                    racc = scatter_recv(k, racc)
            out_ref[...] = acc if racc is None else acc + racc
            for r in rdmas:
                if r is not None:
                    r.wait_send()

    return pl.pallas_call(
        body,
        out_shape=jax.ShapeDtypeStruct((TC, H), jnp.float32),
        in_specs=[
            pl.BlockSpec(memory_space=pltpu.VMEM),
            pl.BlockSpec(memory_space=pltpu.VMEM),
            pl.BlockSpec(memory_space=pltpu.VMEM),
            pl.BlockSpec(memory_space=pl.ANY),
        ],
        out_specs=pl.BlockSpec(memory_space=pltpu.VMEM),
        scratch_shapes=[
            pltpu.VMEM((E_LOC * D, H), jnp.bfloat16),
            pltpu.VMEM((2 * CAP, E_LOC * D), jnp.bfloat16),
            pltpu.VMEM((T, E_LOC), jnp.float32),
            pltpu.VMEM((N_DEV - 1, 1 + CAP, H), jnp.bfloat16),
            pltpu.VMEM((N_DEV - 1, 1 + CAP, H), jnp.bfloat16),
            pltpu.VMEM((E_LOC, D, H), jnp.float32),
            pltpu.SemaphoreType.DMA((N_DEV - 1,)),
            pltpu.SemaphoreType.DMA((N_DEV - 1,)),
            pltpu.SemaphoreType.DMA((E_LOC,)),
        ],
        compiler_params=pltpu.CompilerParams(
            collective_id=None if ABLATE_NO_COMM else 0,
            vmem_limit_bytes=64 * 1024 * 1024,
        ),
    )(x, router_W, route_idx, expert_W)
